# baseline (speedup 1.0000x reference)
import functools
import math

import jax
import jax.numpy as jnp
from jax import lax
from jax.experimental import pallas as pl
from jax.experimental.pallas import tpu as pltpu

BOOTSTRAP_FRAC = 0.4
SEARCH_ITERS = 18


def _body(p_ref, t_ref, out_ref, acc_ref, *, N, NCB, k, scale):
    n = pl.program_id(0)
    cb = pl.program_id(1)

    part = jnp.sum(jnp.log2(p_ref[0]) * t_ref[0], axis=0)

    @pl.when(cb == 0)
    def _init_acc():
        acc_ref[n] = part

    @pl.when(cb > 0)
    def _accum():
        acc_ref[n] += part

    @pl.when((n == N - 1) & (cb == NCB - 1))
    def _select():
        for r in range(N):
            acc_ref[r] = 0.0 - acc_ref[r]

        def count_ge(r, trial):
            vb = lax.bitcast_convert_type(acc_ref[r], jnp.int32)
            return jnp.sum((vb >= trial).astype(jnp.int32))

        def step(i, bits):
            out = []
            for r in range(N):
                trial = bits[r] | (1 << (30 - i))
                out.append(lax.select(count_ge(r, trial) >= k, trial, bits[r]))
            return tuple(out)

        kbits = lax.fori_loop(0, SEARCH_ITERS, step, (jnp.int32(0),) * N)

        rem = 31 - SEARCH_ITERS
        total = jnp.float32(0.0)
        for r in range(N):
            v = acc_ref[r]
            vb = lax.bitcast_convert_type(v, jnp.int32)
            hi = kbits[r] + (1 << rem)
            vk = lax.bitcast_convert_type(
                kbits[r] + (1 << (rem - 1)), jnp.float32
            )
            ge = vb >= hi
            s_ge = jnp.sum(jnp.where(ge, v, 0.0))
            c_ge = jnp.sum(ge.astype(jnp.int32))
            total += s_ge + (k - c_ge).astype(jnp.float32) * vk

        out_ref[0, 0] = total * scale


def kernel(predict, target):
    N, C, H, W = target.shape
    k = int(H * W * BOOTSTRAP_FRAC)
    cblk = 8 if C % 8 == 0 else (4 if C % 4 == 0 else 1)
    ncb = C // cblk

    out = pl.pallas_call(
        functools.partial(
            _body, N=N, NCB=ncb, k=k, scale=math.log(2.0) / (N * k)
        ),
        grid=(N, ncb),
        in_specs=[
            pl.BlockSpec((1, cblk, H, W), lambda n, c: (n, c, 0, 0)),
            pl.BlockSpec((1, cblk, H, W), lambda n, c: (n, c, 0, 0)),
        ],
        out_specs=pl.BlockSpec(memory_space=pltpu.SMEM),
        out_shape=jax.ShapeDtypeStruct((1, 1), jnp.float32),
        scratch_shapes=[pltpu.VMEM((N, H, W), jnp.float32)],
    )(predict, target)
    return out[0, 0]
